# baseline (speedup 1.0000x reference)
import functools

import jax
import jax.numpy as jnp
from jax import lax
from jax.experimental import pallas as pl
from jax.experimental.pallas import tpu as pltpu
from jax.experimental.pallas import tpu_sc as plsc

BATCH = 4096
HIST_LEN = 200
EXPAND_DIM = 64
CHUNK_A = 128
CHUNK_B = HIST_LEN - CHUNK_A


def _rank_body(ts_ref, tst_ref, ia_ref, ib_ref):
    ts = ts_ref[...]
    tsj = tst_ref[...]
    b, h = ts.shape
    tsi = ts[:, None, :]
    jj = jax.lax.broadcasted_iota(jnp.int32, (b, h, h), 1)
    ii = jax.lax.broadcasted_iota(jnp.int32, (b, h, h), 2)
    less = (tsj < tsi) | ((tsj == tsi) & (jj < ii))
    rank = jnp.sum(less.astype(jnp.float32), axis=1)
    base = pl.program_id(0) * b + jax.lax.broadcasted_iota(jnp.int32, (b, h), 0)
    idx = rank.astype(jnp.int32) + base * HIST_LEN
    ia_ref[...] = idx[:, :CHUNK_A]
    ib_ref[...] = idx[:, CHUNK_A:]


def _ranks_to_indices(ts):
    B = 32
    grid = (BATCH // B,)
    ts_t = ts[..., None]
    return pl.pallas_call(
        _rank_body,
        grid=grid,
        in_specs=[
            pl.BlockSpec((B, HIST_LEN), lambda i: (i, 0)),
            pl.BlockSpec((B, HIST_LEN, 1), lambda i: (i, 0, 0)),
        ],
        out_specs=[
            pl.BlockSpec((B, CHUNK_A), lambda i: (i, 0)),
            pl.BlockSpec((B, CHUNK_B), lambda i: (i, 0)),
        ],
        out_shape=[
            jax.ShapeDtypeStruct((BATCH, CHUNK_A), jnp.int32),
            jax.ShapeDtypeStruct((BATCH, CHUNK_B), jnp.int32),
        ],
    )(ts, ts_t)


def _make_scatter():
    info = plsc.get_sparse_core_info()
    nw = info.num_cores * info.num_subcores
    rows_per_w = BATCH // nw
    mesh = plsc.VectorSubcoreMesh(core_axis_name="c", subcore_axis_name="s")

    @functools.partial(
        pl.kernel,
        mesh=mesh,
        out_type=jax.ShapeDtypeStruct((BATCH * HIST_LEN, EXPAND_DIM),
                                      jnp.float32),
        scratch_types=[
            pltpu.VMEM((HIST_LEN, EXPAND_DIM), jnp.float32),
            pltpu.VMEM((rows_per_w, CHUNK_A), jnp.int32),
            pltpu.VMEM((rows_per_w, CHUNK_B), jnp.int32),
            pltpu.SemaphoreType.DMA,
        ],
        compiler_params=pltpu.CompilerParams(use_tc_tiling_on_sc=False),
    )
    def scatter(table_hbm, ia_hbm, ib_hbm, out_hbm, table_v, ia_v, ib_v, sem):
        wid = lax.axis_index("s") * info.num_cores + lax.axis_index("c")
        base = wid * rows_per_w
        pltpu.sync_copy(table_hbm, table_v)
        pltpu.sync_copy(ia_hbm.at[pl.ds(base, rows_per_w)], ia_v)
        pltpu.sync_copy(ib_hbm.at[pl.ds(base, rows_per_w)], ib_v)

        def body(r):
            ca = pltpu.async_copy(table_v.at[pl.ds(0, CHUNK_A)],
                                  out_hbm.at[ia_v.at[r]], sem)
            cb = pltpu.async_copy(table_v.at[pl.ds(CHUNK_A, CHUNK_B)],
                                  out_hbm.at[ib_v.at[r]], sem)
            ca.wait()
            cb.wait()

        pl.loop(0, rows_per_w)(body)

    return scatter


_scatter = _make_scatter()


@jax.jit
def kernel(ts, pos_embeddings):
    ia, ib = _ranks_to_indices(ts)
    out = _scatter(pos_embeddings, ia, ib)
    return out.reshape(BATCH, HIST_LEN, EXPAND_DIM)


# speedup vs baseline: 2.0752x; 2.0752x over previous
import jax
import jax.numpy as jnp
from jax.experimental import pallas as pl

BATCH = 4096
HIST_LEN = 200
EXPAND_DIM = 64
BB = 128
IB = 40
N_IB = HIST_LEN // IB


def _body(tst_ref, et_ref, out_ref):
    ib = pl.program_id(1)
    tst = tst_ref[...]
    h, nb = tst.shape
    tsi = tst_ref[pl.ds(ib * IB, IB), :][None, :, :]
    tsj = tst[:, None, :]
    d = tsj - tsi
    jj = jax.lax.broadcasted_iota(jnp.int32, (h, IB, nb), 0)
    ii = jax.lax.broadcasted_iota(jnp.int32, (h, IB, nb), 1) + ib * IB
    less = (d < 0) | ((d == 0) & (jj < ii))
    rank = jnp.sum(less.astype(jnp.float32), axis=0)
    kk = jax.lax.broadcasted_iota(jnp.int32, (h, IB, nb), 0).astype(jnp.float32)
    oh = (rank[None, :, :] == kk).astype(jnp.float32)
    et = jnp.broadcast_to(et_ref[0][None, :, :], (h, EXPAND_DIM, IB))
    part = jax.lax.dot_general(
        et, oh, (((2,), (1,)), ((0,), (0,))),
        preferred_element_type=jnp.float32,
    )

    @pl.when(ib == 0)
    def _init():
        out_ref[...] = part

    @pl.when(ib > 0)
    def _acc():
        out_ref[...] += part


@jax.jit
def kernel(ts, pos_embeddings):
    ts_t = ts.T
    et = pos_embeddings.T.reshape(EXPAND_DIM, N_IB, IB).transpose(1, 0, 2)
    grid = (BATCH // BB, N_IB)
    out_t = pl.pallas_call(
        _body,
        grid=grid,
        in_specs=[
            pl.BlockSpec((HIST_LEN, BB), lambda bb, ib: (0, bb)),
            pl.BlockSpec((1, EXPAND_DIM, IB), lambda bb, ib: (ib, 0, 0)),
        ],
        out_specs=pl.BlockSpec((HIST_LEN, EXPAND_DIM, BB),
                               lambda bb, ib: (0, 0, bb)),
        out_shape=jax.ShapeDtypeStruct((HIST_LEN, EXPAND_DIM, BATCH),
                                       jnp.float32),
    )(ts_t, et)
    return out_t.transpose(2, 0, 1)


# speedup vs baseline: 4.2539x; 2.0498x over previous
import jax
import jax.numpy as jnp
from jax.experimental import pallas as pl

BATCH = 4096
HIST_LEN = 200
EXPAND_DIM = 64
BB = 128
IB = 200
N_IB = HIST_LEN // IB


def _body(tst_ref, et_ref, out_ref):
    ib = pl.program_id(1)
    tst = tst_ref[...]
    h, nb = tst.shape
    tsi = tst_ref[pl.ds(ib * IB, IB), :][None, :, :]
    tsj = tst[:, None, :]
    d = tsj - tsi
    jj = jax.lax.broadcasted_iota(jnp.int32, (h, IB, nb), 0)
    ii = jax.lax.broadcasted_iota(jnp.int32, (h, IB, nb), 1) + ib * IB
    less = (d < 0) | ((d == 0) & (jj < ii))
    rank = jnp.sum(less.astype(jnp.float32), axis=0)
    kk = jax.lax.broadcasted_iota(jnp.int32, (h, IB, nb), 0).astype(jnp.float32)
    oh = (rank[None, :, :] == kk).astype(jnp.float32)
    et = jnp.broadcast_to(et_ref[0][None, :, :], (h, EXPAND_DIM, IB))
    part = jax.lax.dot_general(
        et, oh, (((2,), (1,)), ((0,), (0,))),
        preferred_element_type=jnp.float32,
    )

    @pl.when(ib == 0)
    def _init():
        out_ref[...] = part

    @pl.when(ib > 0)
    def _acc():
        out_ref[...] += part


@jax.jit
def kernel(ts, pos_embeddings):
    ts_t = ts.T
    et = pos_embeddings.T.reshape(EXPAND_DIM, N_IB, IB).transpose(1, 0, 2)
    grid = (BATCH // BB, N_IB)
    out_t = pl.pallas_call(
        _body,
        grid=grid,
        in_specs=[
            pl.BlockSpec((HIST_LEN, BB), lambda bb, ib: (0, bb)),
            pl.BlockSpec((1, EXPAND_DIM, IB), lambda bb, ib: (ib, 0, 0)),
        ],
        out_specs=pl.BlockSpec((HIST_LEN, EXPAND_DIM, BB),
                               lambda bb, ib: (0, 0, bb)),
        out_shape=jax.ShapeDtypeStruct((HIST_LEN, EXPAND_DIM, BATCH),
                                       jnp.float32),
    )(ts_t, et)
    return out_t.transpose(2, 0, 1)
